# baseline (speedup 1.0000x reference)
import functools

import jax
import jax.numpy as jnp
from jax import lax
from jax.experimental import pallas as pl
from jax.experimental.pallas import tpu as pltpu
from jax.experimental.pallas import tpu_sc as plsc

NNODE = 10000
NEDGE = 320000
NC = 2
NS = 16
NW = NC * NS
CHUNK = 128
CH = 80
EPAD = NW * CH * CHUNK
NPAD = 10112
TRASH = NNODE
ZR = NPAD // NS
W16 = 16
NBUF = 4
RING = 8

_SC_PARAMS = pltpu.CompilerParams(use_tc_tiling_on_sc=False)


def _sc_aggregate(g, src3, dst3, zrows, do_gather=True):
    mesh = plsc.VectorSubcoreMesh(core_axis_name="c", subcore_axis_name="s")

    @functools.partial(
        pl.kernel,
        out_type=jax.ShapeDtypeStruct((NC, NPAD, W16), jnp.float32),
        mesh=mesh,
        scratch_types=[
            pltpu.VMEM((CH, CHUNK), jnp.int32),
            pltpu.VMEM((CH, CHUNK), jnp.int32),
            pltpu.VMEM((RING, CHUNK, W16), jnp.float32),
            pltpu.VMEM_SHARED((NPAD, W16), jnp.float32),
            pltpu.SemaphoreType.DMA,
            pltpu.SemaphoreType.DMA,
        ],
        compiler_params=_SC_PARAMS,
    )
    def k(g_hbm, src_hbm, dst_hbm, z_hbm, out_hbm, src_v, dst_v, rows_v, acc,
          gsem, ssem):
        c = lax.axis_index("c")
        s = lax.axis_index("s")
        wid = s * NC + c

        pltpu.sync_copy(src_hbm.at[wid], src_v)
        pltpu.sync_copy(dst_hbm.at[wid], dst_v)
        if not do_gather:
            pltpu.sync_copy(g_hbm.at[pl.ds(0, CHUNK)], rows_v.at[0])
        pltpu.sync_copy(z_hbm, acc.at[pl.ds(s * ZR, ZR)])
        plsc.subcore_barrier()

        if do_gather:
            for b in range(NBUF):
                pltpu.async_copy(g_hbm.at[src_v.at[b]], rows_v.at[b], gsem)
            for j in range(CH):
                b = j % RING
                pltpu.make_async_copy(g_hbm.at[src_v.at[j]],
                                      rows_v.at[b], gsem).wait()
                pltpu.async_copy(rows_v.at[b], acc.at[dst_v.at[j]], ssem,
                                 add=True)
                jn = j + NBUF
                if jn < CH:
                    jo = jn - RING
                    if jo >= 0:
                        pltpu.make_async_copy(rows_v.at[jo % RING],
                                              acc.at[dst_v.at[jo]],
                                              ssem).wait()
                    pltpu.async_copy(g_hbm.at[src_v.at[jn]],
                                     rows_v.at[jn % RING], gsem)
            for j in range(max(CH - RING, 0), CH):
                pltpu.make_async_copy(rows_v.at[j % RING],
                                      acc.at[dst_v.at[j]], ssem).wait()
        else:
            for j in range(CH):
                pltpu.async_copy(rows_v.at[0], acc.at[dst_v.at[j]], ssem,
                                 add=True)
            for j in range(CH):
                pltpu.make_async_copy(rows_v.at[0], acc.at[dst_v.at[j]],
                                      ssem).wait()

        plsc.subcore_barrier()
        pltpu.sync_copy(acc.at[pl.ds(s * ZR, ZR)],
                        out_hbm.at[c, pl.ds(s * ZR, ZR)])

    return k(g, src3, dst3, zrows)


def _tc(body, n_out, *args):
    outs = tuple(jax.ShapeDtypeStruct((NPAD, w), jnp.float32) for w in n_out)
    return pl.pallas_call(body, out_shape=outs)(*args)


def _mm(a, b):
    return jnp.dot(a, b, preferred_element_type=jnp.float32)


def _pad16(w):
    return jnp.pad(w, ((0, W16 - w.shape[0]), (0, W16 - w.shape[1])))


def kernel(x, edge_index, W1, W2, W3, LW1, Lb1, LW2, Lb2, LW3, Lb3, TW1, TW2, TW3):
    f32 = jnp.float32
    xp = jnp.pad(x, ((0, NPAD - NNODE), (0, 0)))
    src = jnp.concatenate([edge_index[0],
                           jnp.zeros((EPAD - NEDGE,), edge_index.dtype)])
    dst = jnp.concatenate([edge_index[1],
                           jnp.full((EPAD - NEDGE,), TRASH, edge_index.dtype)])
    src3 = src.reshape(NW, CH, CHUNK)
    dst3 = dst.reshape(NW, CH, CHUNK)
    ones_blk = jnp.ones((NPAD, W16), f32)
    zrows = jnp.zeros((ZR, W16), f32)
    W2p, W3p = _pad16(W2), _pad16(W3)
    LW1p, TW1p = _pad16(LW1), _pad16(TW1)
    LW2p, TW2p = _pad16(LW2), _pad16(TW2)
    b1 = jnp.pad(Lb1, (0, W16 - Lb1.shape[0])).reshape(1, W16)
    b2 = jnp.pad(Lb2, (0, W16 - Lb2.shape[0])).reshape(1, W16)
    b3 = Lb3.reshape(1, -1)

    degp = _sc_aggregate(ones_blk, src3, dst3, zrows, do_gather=False)

    def tc_h1(x_r, w_r, o_r):
        o_r[...] = _mm(x_r[...], w_r[...])
    (h1,) = _tc(tc_h1, [16], xp, W1)

    def tc_g1(d_r, h_r, dinv_r, g_r):
        dinv = lax.rsqrt(d_r[0, :, 0:1] + d_r[1, :, 0:1] + 1.0)
        dinv_r[...] = dinv
        g_r[...] = dinv * h_r[...]
    dinv, g1 = _tc(tc_g1, [1, 16], degp, h1)

    def tc_enc(p_r, g_r, dinv_r, w_r, o_r):
        dinv = dinv_r[...]
        c = jax.nn.relu(dinv * (p_r[0] + p_r[1] + g_r[...]))
        o_r[...] = dinv * _mm(c, w_r[...])

    p1 = _sc_aggregate(g1, src3, dst3, zrows)
    (g2,) = _tc(tc_enc, [16], p1, g1, dinv, W2p)
    p2 = _sc_aggregate(g2, src3, dst3, zrows)
    (g3,) = _tc(tc_enc, [16], p2, g2, dinv, W3p)
    p3 = _sc_aggregate(g3, src3, dst3, zrows)

    def tc_hid(p_r, g_r, dinv_r, h_r, g4_r):
        dinv = dinv_r[...]
        h3 = dinv * (p_r[0] + p_r[1] + g_r[...])
        h_r[...] = h3
        g4_r[...] = dinv * h3
    h3, g4 = _tc(tc_hid, [16, 16], p3, g3, dinv)

    def tc_dec(p_r, g_r, dinv_r, yp_r, lw_r, b_r, tw_r, y_r, gn_r, *, relu):
        dinv = dinv_r[...]
        sh = dinv * (p_r[0] + p_r[1] + g_r[...])
        t = 2.0 * (_mm(yp_r[...], lw_r[...]) + b_r[...]) - _mm(sh, tw_r[...])
        y = jax.nn.relu(t) if relu else t
        y_r[...] = y
        gn_r[...] = dinv * y

    p4 = _sc_aggregate(g4, src3, dst3, zrows)
    y1, g5 = _tc(functools.partial(tc_dec, relu=True), [16, 16],
                 p4, g4, dinv, h3, LW1p, b1, TW1p)
    p5 = _sc_aggregate(g5, src3, dst3, zrows)
    y2, g6 = _tc(functools.partial(tc_dec, relu=True), [16, 16],
                 p5, g5, dinv, y1, LW2p, b2, TW2p)
    p6 = _sc_aggregate(g6, src3, dst3, zrows)

    def tc_out(p_r, g_r, dinv_r, yp_r, lw_r, b_r, tw_r, y_r):
        sh = dinv_r[...] * (p_r[0] + p_r[1] + g_r[...])
        y_r[...] = (2.0 * (_mm(yp_r[...], lw_r[...]) + b_r[...])
                    - _mm(sh, tw_r[...]))
    (y,) = _tc(tc_out, [128], p6, g6, dinv, y2, LW3, b3, TW3)

    return (h3[:NNODE, :4], y[:NNODE])


# speedup vs baseline: 1.0265x; 1.0265x over previous
import functools

import jax
import jax.numpy as jnp
from jax import lax
from jax.experimental import pallas as pl
from jax.experimental.pallas import tpu as pltpu
from jax.experimental.pallas import tpu_sc as plsc

NNODE = 10000
NEDGE = 320000
NC = 2
NS = 16
NW = NC * NS
CHUNK = 256
CH = 40
EPAD = NW * CH * CHUNK
NPAD = 10112
TRASH = NNODE
ZR = NPAD // NS
W16 = 16
NBUF = 4
RING = 8

_SC_PARAMS = pltpu.CompilerParams(use_tc_tiling_on_sc=False)


def _sc_aggregate(g, src3, dst3, zrows, do_gather=True):
    mesh = plsc.VectorSubcoreMesh(core_axis_name="c", subcore_axis_name="s")

    @functools.partial(
        pl.kernel,
        out_type=jax.ShapeDtypeStruct((NC, NPAD, W16), jnp.float32),
        mesh=mesh,
        scratch_types=[
            pltpu.VMEM((CH, CHUNK), jnp.int32),
            pltpu.VMEM((CH, CHUNK), jnp.int32),
            pltpu.VMEM((RING, CHUNK, W16), jnp.float32),
            pltpu.VMEM_SHARED((NPAD, W16), jnp.float32),
            pltpu.SemaphoreType.DMA,
            pltpu.SemaphoreType.DMA,
        ],
        compiler_params=_SC_PARAMS,
    )
    def k(g_hbm, src_hbm, dst_hbm, z_hbm, out_hbm, src_v, dst_v, rows_v, acc,
          gsem, ssem):
        c = lax.axis_index("c")
        s = lax.axis_index("s")
        wid = s * NC + c

        pltpu.sync_copy(src_hbm.at[wid], src_v)
        pltpu.sync_copy(dst_hbm.at[wid], dst_v)
        if not do_gather:
            pltpu.sync_copy(g_hbm.at[pl.ds(0, CHUNK)], rows_v.at[0])
        pltpu.sync_copy(z_hbm, acc.at[pl.ds(s * ZR, ZR)])
        plsc.subcore_barrier()

        if do_gather:
            for b in range(NBUF):
                pltpu.async_copy(g_hbm.at[src_v.at[b]], rows_v.at[b], gsem)
            for j in range(CH):
                b = j % RING
                pltpu.make_async_copy(g_hbm.at[src_v.at[j]],
                                      rows_v.at[b], gsem).wait()
                pltpu.async_copy(rows_v.at[b], acc.at[dst_v.at[j]], ssem,
                                 add=True)
                jn = j + NBUF
                if jn < CH:
                    jo = jn - RING
                    if jo >= 0:
                        pltpu.make_async_copy(rows_v.at[jo % RING],
                                              acc.at[dst_v.at[jo]],
                                              ssem).wait()
                    pltpu.async_copy(g_hbm.at[src_v.at[jn]],
                                     rows_v.at[jn % RING], gsem)
            for j in range(max(CH - RING, 0), CH):
                pltpu.make_async_copy(rows_v.at[j % RING],
                                      acc.at[dst_v.at[j]], ssem).wait()
        else:
            for j in range(CH):
                pltpu.async_copy(rows_v.at[0], acc.at[dst_v.at[j]], ssem,
                                 add=True)
            for j in range(CH):
                pltpu.make_async_copy(rows_v.at[0], acc.at[dst_v.at[j]],
                                      ssem).wait()

        plsc.subcore_barrier()
        pltpu.sync_copy(acc.at[pl.ds(s * ZR, ZR)],
                        out_hbm.at[c, pl.ds(s * ZR, ZR)])

    return k(g, src3, dst3, zrows)


def _tc(body, n_out, *args):
    outs = tuple(jax.ShapeDtypeStruct((NPAD, w), jnp.float32) for w in n_out)
    return pl.pallas_call(body, out_shape=outs)(*args)


def _mm(a, b):
    return jnp.dot(a, b, preferred_element_type=jnp.float32)


def _pad16(w):
    return jnp.pad(w, ((0, W16 - w.shape[0]), (0, W16 - w.shape[1])))


def kernel(x, edge_index, W1, W2, W3, LW1, Lb1, LW2, Lb2, LW3, Lb3, TW1, TW2, TW3):
    f32 = jnp.float32
    xp = jnp.pad(x, ((0, NPAD - NNODE), (0, 0)))
    src = jnp.concatenate([edge_index[0],
                           jnp.zeros((EPAD - NEDGE,), edge_index.dtype)])
    dst = jnp.concatenate([edge_index[1],
                           jnp.full((EPAD - NEDGE,), TRASH, edge_index.dtype)])
    src3 = src.reshape(NW, CH, CHUNK)
    dst3 = dst.reshape(NW, CH, CHUNK)
    ones_blk = jnp.ones((NPAD, W16), f32)
    zrows = jnp.zeros((ZR, W16), f32)
    W2p, W3p = _pad16(W2), _pad16(W3)
    LW1p, TW1p = _pad16(LW1), _pad16(TW1)
    LW2p, TW2p = _pad16(LW2), _pad16(TW2)
    b1 = jnp.pad(Lb1, (0, W16 - Lb1.shape[0])).reshape(1, W16)
    b2 = jnp.pad(Lb2, (0, W16 - Lb2.shape[0])).reshape(1, W16)
    b3 = Lb3.reshape(1, -1)

    degp = _sc_aggregate(ones_blk, src3, dst3, zrows, do_gather=False)

    def tc_h1(x_r, w_r, o_r):
        o_r[...] = _mm(x_r[...], w_r[...])
    (h1,) = _tc(tc_h1, [16], xp, W1)

    def tc_g1(d_r, h_r, dinv_r, g_r):
        dinv = lax.rsqrt(d_r[0, :, 0:1] + d_r[1, :, 0:1] + 1.0)
        dinv_r[...] = dinv
        g_r[...] = dinv * h_r[...]
    dinv, g1 = _tc(tc_g1, [1, 16], degp, h1)

    def tc_enc(p_r, g_r, dinv_r, w_r, o_r):
        dinv = dinv_r[...]
        c = jax.nn.relu(dinv * (p_r[0] + p_r[1] + g_r[...]))
        o_r[...] = dinv * _mm(c, w_r[...])

    p1 = _sc_aggregate(g1, src3, dst3, zrows)
    (g2,) = _tc(tc_enc, [16], p1, g1, dinv, W2p)
    p2 = _sc_aggregate(g2, src3, dst3, zrows)
    (g3,) = _tc(tc_enc, [16], p2, g2, dinv, W3p)
    p3 = _sc_aggregate(g3, src3, dst3, zrows)

    def tc_hid(p_r, g_r, dinv_r, h_r, g4_r):
        dinv = dinv_r[...]
        h3 = dinv * (p_r[0] + p_r[1] + g_r[...])
        h_r[...] = h3
        g4_r[...] = dinv * h3
    h3, g4 = _tc(tc_hid, [16, 16], p3, g3, dinv)

    def tc_dec(p_r, g_r, dinv_r, yp_r, lw_r, b_r, tw_r, y_r, gn_r, *, relu):
        dinv = dinv_r[...]
        sh = dinv * (p_r[0] + p_r[1] + g_r[...])
        t = 2.0 * (_mm(yp_r[...], lw_r[...]) + b_r[...]) - _mm(sh, tw_r[...])
        y = jax.nn.relu(t) if relu else t
        y_r[...] = y
        gn_r[...] = dinv * y

    p4 = _sc_aggregate(g4, src3, dst3, zrows)
    y1, g5 = _tc(functools.partial(tc_dec, relu=True), [16, 16],
                 p4, g4, dinv, h3, LW1p, b1, TW1p)
    p5 = _sc_aggregate(g5, src3, dst3, zrows)
    y2, g6 = _tc(functools.partial(tc_dec, relu=True), [16, 16],
                 p5, g5, dinv, y1, LW2p, b2, TW2p)
    p6 = _sc_aggregate(g6, src3, dst3, zrows)

    def tc_out(p_r, g_r, dinv_r, yp_r, lw_r, b_r, tw_r, y_r):
        sh = dinv_r[...] * (p_r[0] + p_r[1] + g_r[...])
        y_r[...] = (2.0 * (_mm(yp_r[...], lw_r[...]) + b_r[...])
                    - _mm(sh, tw_r[...]))
    (y,) = _tc(tc_out, [128], p6, g6, dinv, y2, LW3, b3, TW3)

    return (h3[:NNODE, :4], y[:NNODE])


# speedup vs baseline: 1.6719x; 1.6287x over previous
import functools

import jax
import jax.numpy as jnp
from jax import lax
from jax.experimental import pallas as pl
from jax.experimental.pallas import tpu as pltpu
from jax.experimental.pallas import tpu_sc as plsc

NNODE = 10000
NEDGE = 320000
NC = 2
NS = 16
NW = NC * NS
CHUNK = 256
CH = 40
EPAD = NW * CH * CHUNK
NPAD = 10112
TRASH = NNODE
ZR = NPAD // NS
W16 = 16
NBUF = 4
RING = 8

_SC_PARAMS = pltpu.CompilerParams(use_tc_tiling_on_sc=False)


def _sc_aggregate(g, src3, dst3, zrows, do_gather=True):
    mesh = plsc.VectorSubcoreMesh(core_axis_name="c", subcore_axis_name="s")

    @functools.partial(
        pl.kernel,
        out_type=jax.ShapeDtypeStruct((NC, NPAD, W16), jnp.float32),
        mesh=mesh,
        scratch_types=[
            pltpu.VMEM((CH, CHUNK), jnp.int32),
            pltpu.VMEM((CH, CHUNK), jnp.int32),
            pltpu.VMEM((RING, CHUNK, W16), jnp.float32),
            pltpu.VMEM_SHARED((NPAD, W16), jnp.float32),
            pltpu.VMEM_SHARED((NPAD, W16), jnp.float32),
            pltpu.SemaphoreType.DMA,
            pltpu.SemaphoreType.DMA,
        ],
        compiler_params=_SC_PARAMS,
    )
    def k(g_hbm, src_hbm, dst_hbm, z_hbm, out_hbm, src_v, dst_v, rows_v, acc,
          g_sh, gsem, ssem):
        c = lax.axis_index("c")
        s = lax.axis_index("s")
        wid = s * NC + c

        pltpu.sync_copy(src_hbm.at[wid], src_v)
        pltpu.sync_copy(dst_hbm.at[wid], dst_v)
        if do_gather:
            pltpu.sync_copy(g_hbm.at[pl.ds(s * ZR, ZR)],
                            g_sh.at[pl.ds(s * ZR, ZR)])
        else:
            pltpu.sync_copy(g_hbm.at[pl.ds(0, CHUNK)], rows_v.at[0])
        pltpu.sync_copy(z_hbm, acc.at[pl.ds(s * ZR, ZR)])
        plsc.subcore_barrier()

        if do_gather:
            for b in range(NBUF):
                pltpu.async_copy(g_sh.at[src_v.at[b]], rows_v.at[b], gsem)
            for j in range(CH):
                b = j % RING
                pltpu.make_async_copy(g_sh.at[src_v.at[j]],
                                      rows_v.at[b], gsem).wait()
                pltpu.async_copy(rows_v.at[b], acc.at[dst_v.at[j]], ssem,
                                 add=True)
                jn = j + NBUF
                if jn < CH:
                    jo = jn - RING
                    if jo >= 0:
                        pltpu.make_async_copy(rows_v.at[jo % RING],
                                              acc.at[dst_v.at[jo]],
                                              ssem).wait()
                    pltpu.async_copy(g_sh.at[src_v.at[jn]],
                                     rows_v.at[jn % RING], gsem)
            for j in range(max(CH - RING, 0), CH):
                pltpu.make_async_copy(rows_v.at[j % RING],
                                      acc.at[dst_v.at[j]], ssem).wait()
        else:
            for j in range(CH):
                pltpu.async_copy(rows_v.at[0], acc.at[dst_v.at[j]], ssem,
                                 add=True)
            for j in range(CH):
                pltpu.make_async_copy(rows_v.at[0], acc.at[dst_v.at[j]],
                                      ssem).wait()

        plsc.subcore_barrier()
        pltpu.sync_copy(acc.at[pl.ds(s * ZR, ZR)],
                        out_hbm.at[c, pl.ds(s * ZR, ZR)])

    return k(g, src3, dst3, zrows)


def _tc(body, n_out, *args):
    outs = tuple(jax.ShapeDtypeStruct((NPAD, w), jnp.float32) for w in n_out)
    return pl.pallas_call(body, out_shape=outs)(*args)


def _mm(a, b):
    return jnp.dot(a, b, preferred_element_type=jnp.float32)


def _pad16(w):
    return jnp.pad(w, ((0, W16 - w.shape[0]), (0, W16 - w.shape[1])))


def kernel(x, edge_index, W1, W2, W3, LW1, Lb1, LW2, Lb2, LW3, Lb3, TW1, TW2, TW3):
    f32 = jnp.float32
    xp = jnp.pad(x, ((0, NPAD - NNODE), (0, 0)))
    src = jnp.concatenate([edge_index[0],
                           jnp.zeros((EPAD - NEDGE,), edge_index.dtype)])
    dst = jnp.concatenate([edge_index[1],
                           jnp.full((EPAD - NEDGE,), TRASH, edge_index.dtype)])
    src3 = src.reshape(NW, CH, CHUNK)
    dst3 = dst.reshape(NW, CH, CHUNK)
    ones_blk = jnp.ones((NPAD, W16), f32)
    zrows = jnp.zeros((ZR, W16), f32)
    W2p, W3p = _pad16(W2), _pad16(W3)
    LW1p, TW1p = _pad16(LW1), _pad16(TW1)
    LW2p, TW2p = _pad16(LW2), _pad16(TW2)
    b1 = jnp.pad(Lb1, (0, W16 - Lb1.shape[0])).reshape(1, W16)
    b2 = jnp.pad(Lb2, (0, W16 - Lb2.shape[0])).reshape(1, W16)
    b3 = Lb3.reshape(1, -1)

    degp = _sc_aggregate(ones_blk, src3, dst3, zrows, do_gather=False)

    def tc_h1(x_r, w_r, o_r):
        o_r[...] = _mm(x_r[...], w_r[...])
    (h1,) = _tc(tc_h1, [16], xp, W1)

    def tc_g1(d_r, h_r, dinv_r, g_r):
        dinv = lax.rsqrt(d_r[0, :, 0:1] + d_r[1, :, 0:1] + 1.0)
        dinv_r[...] = dinv
        g_r[...] = dinv * h_r[...]
    dinv, g1 = _tc(tc_g1, [1, 16], degp, h1)

    def tc_enc(p_r, g_r, dinv_r, w_r, o_r):
        dinv = dinv_r[...]
        c = jax.nn.relu(dinv * (p_r[0] + p_r[1] + g_r[...]))
        o_r[...] = dinv * _mm(c, w_r[...])

    p1 = _sc_aggregate(g1, src3, dst3, zrows)
    (g2,) = _tc(tc_enc, [16], p1, g1, dinv, W2p)
    p2 = _sc_aggregate(g2, src3, dst3, zrows)
    (g3,) = _tc(tc_enc, [16], p2, g2, dinv, W3p)
    p3 = _sc_aggregate(g3, src3, dst3, zrows)

    def tc_hid(p_r, g_r, dinv_r, h_r, g4_r):
        dinv = dinv_r[...]
        h3 = dinv * (p_r[0] + p_r[1] + g_r[...])
        h_r[...] = h3
        g4_r[...] = dinv * h3
    h3, g4 = _tc(tc_hid, [16, 16], p3, g3, dinv)

    def tc_dec(p_r, g_r, dinv_r, yp_r, lw_r, b_r, tw_r, y_r, gn_r, *, relu):
        dinv = dinv_r[...]
        sh = dinv * (p_r[0] + p_r[1] + g_r[...])
        t = 2.0 * (_mm(yp_r[...], lw_r[...]) + b_r[...]) - _mm(sh, tw_r[...])
        y = jax.nn.relu(t) if relu else t
        y_r[...] = y
        gn_r[...] = dinv * y

    p4 = _sc_aggregate(g4, src3, dst3, zrows)
    y1, g5 = _tc(functools.partial(tc_dec, relu=True), [16, 16],
                 p4, g4, dinv, h3, LW1p, b1, TW1p)
    p5 = _sc_aggregate(g5, src3, dst3, zrows)
    y2, g6 = _tc(functools.partial(tc_dec, relu=True), [16, 16],
                 p5, g5, dinv, y1, LW2p, b2, TW2p)
    p6 = _sc_aggregate(g6, src3, dst3, zrows)

    def tc_out(p_r, g_r, dinv_r, yp_r, lw_r, b_r, tw_r, y_r):
        sh = dinv_r[...] * (p_r[0] + p_r[1] + g_r[...])
        y_r[...] = (2.0 * (_mm(yp_r[...], lw_r[...]) + b_r[...])
                    - _mm(sh, tw_r[...]))
    (y,) = _tc(tc_out, [128], p6, g6, dinv, y2, LW3, b3, TW3)

    return (h3[:NNODE, :4], y[:NNODE])


# speedup vs baseline: 1.6725x; 1.0004x over previous
import functools

import jax
import jax.numpy as jnp
from jax import lax
from jax.experimental import pallas as pl
from jax.experimental.pallas import tpu as pltpu
from jax.experimental.pallas import tpu_sc as plsc

NNODE = 10000
NEDGE = 320000
NC = 2
NS = 16
NW = NC * NS
CHUNK = 256
CH = 40
EPAD = NW * CH * CHUNK
NPAD = 10112
TRASH = NNODE
ZR = NPAD // NS
W16 = 16
NBUF = 4
RING = 8

_SC_PARAMS = pltpu.CompilerParams(use_tc_tiling_on_sc=False)


def _sc_aggregate(g, src3, dst3, zrows, width, do_gather=True):
    mesh = plsc.VectorSubcoreMesh(core_axis_name="c", subcore_axis_name="s")

    @functools.partial(
        pl.kernel,
        out_type=jax.ShapeDtypeStruct((NC, NPAD, width), jnp.float32),
        mesh=mesh,
        scratch_types=[
            pltpu.VMEM((CH, CHUNK), jnp.int32),
            pltpu.VMEM((CH, CHUNK), jnp.int32),
            pltpu.VMEM((RING, CHUNK, width), jnp.float32),
            pltpu.VMEM_SHARED((NPAD, width), jnp.float32),
            pltpu.VMEM_SHARED((NPAD, width), jnp.float32),
            pltpu.SemaphoreType.DMA,
            pltpu.SemaphoreType.DMA,
        ],
        compiler_params=_SC_PARAMS,
    )
    def k(g_hbm, src_hbm, dst_hbm, z_hbm, out_hbm, src_v, dst_v, rows_v, acc,
          g_sh, gsem, ssem):
        c = lax.axis_index("c")
        s = lax.axis_index("s")
        wid = s * NC + c

        pltpu.sync_copy(src_hbm.at[wid], src_v)
        pltpu.sync_copy(dst_hbm.at[wid], dst_v)
        if do_gather:
            pltpu.sync_copy(g_hbm.at[pl.ds(s * ZR, ZR)],
                            g_sh.at[pl.ds(s * ZR, ZR)])
        else:
            pltpu.sync_copy(g_hbm.at[pl.ds(0, CHUNK)], rows_v.at[0])
        pltpu.sync_copy(z_hbm, acc.at[pl.ds(s * ZR, ZR)])
        plsc.subcore_barrier()

        if do_gather:
            for b in range(NBUF):
                pltpu.async_copy(g_sh.at[src_v.at[b]], rows_v.at[b], gsem)
            for j in range(CH):
                b = j % RING
                pltpu.make_async_copy(g_sh.at[src_v.at[j]],
                                      rows_v.at[b], gsem).wait()
                pltpu.async_copy(rows_v.at[b], acc.at[dst_v.at[j]], ssem,
                                 add=True)
                jn = j + NBUF
                if jn < CH:
                    jo = jn - RING
                    if jo >= 0:
                        pltpu.make_async_copy(rows_v.at[jo % RING],
                                              acc.at[dst_v.at[jo]],
                                              ssem).wait()
                    pltpu.async_copy(g_sh.at[src_v.at[jn]],
                                     rows_v.at[jn % RING], gsem)
            for j in range(max(CH - RING, 0), CH):
                pltpu.make_async_copy(rows_v.at[j % RING],
                                      acc.at[dst_v.at[j]], ssem).wait()
        else:
            for j in range(CH):
                pltpu.async_copy(rows_v.at[0], acc.at[dst_v.at[j]], ssem,
                                 add=True)
            for j in range(CH):
                pltpu.make_async_copy(rows_v.at[0], acc.at[dst_v.at[j]],
                                      ssem).wait()

        plsc.subcore_barrier()
        pltpu.sync_copy(acc.at[pl.ds(s * ZR, ZR)],
                        out_hbm.at[c, pl.ds(s * ZR, ZR)])

    return k(g, src3, dst3, zrows)


def _tc(body, n_out, *args):
    outs = tuple(jax.ShapeDtypeStruct((NPAD, w), jnp.float32) for w in n_out)
    return pl.pallas_call(body, out_shape=outs)(*args)


def _mm(a, b):
    return jnp.dot(a, b, preferred_element_type=jnp.float32)


def _pad16(w):
    return jnp.pad(w, ((0, W16 - w.shape[0]), (0, W16 - w.shape[1])))


def kernel(x, edge_index, W1, W2, W3, LW1, Lb1, LW2, Lb2, LW3, Lb3, TW1, TW2, TW3):
    f32 = jnp.float32
    xp = jnp.pad(x, ((0, NPAD - NNODE), (0, 0)))
    src = jnp.concatenate([edge_index[0],
                           jnp.zeros((EPAD - NEDGE,), edge_index.dtype)])
    dst = jnp.concatenate([edge_index[1],
                           jnp.full((EPAD - NEDGE,), TRASH, edge_index.dtype)])
    src3 = src.reshape(NW, CH, CHUNK)
    dst3 = dst.reshape(NW, CH, CHUNK)
    ones_blk = jnp.ones((NPAD, W16), f32)
    zw = {w: jnp.zeros((ZR, w), f32) for w in (W16,)}
    W2p, W3p = _pad16(W2), _pad16(W3)
    LW1p, TW1p = _pad16(LW1), _pad16(TW1)
    LW2p, TW2p = _pad16(LW2), _pad16(TW2)
    b1 = jnp.pad(Lb1, (0, W16 - Lb1.shape[0])).reshape(1, W16)
    b2 = jnp.pad(Lb2, (0, W16 - Lb2.shape[0])).reshape(1, W16)
    b3 = Lb3.reshape(1, -1)

    degp = _sc_aggregate(ones_blk, src3, dst3, zw[W16], W16, do_gather=False)

    def tc_h1(x_r, w_r, o_r):
        o_r[...] = _mm(x_r[...], w_r[...])
    (h1,) = _tc(tc_h1, [16], xp, W1)

    def tc_g1(d_r, h_r, dinv_r, g_r):
        dinv = lax.rsqrt(d_r[0, :, 0:1] + d_r[1, :, 0:1] + 1.0)
        dinv_r[...] = dinv
        g_r[...] = dinv * h_r[...]
    dinv, g1 = _tc(tc_g1, [1, 16], degp, h1)

    def tc_enc(p_r, g_r, dinv_r, w_r, o_r):
        dinv = dinv_r[...]
        c = jax.nn.relu(dinv * (p_r[0] + p_r[1] + g_r[...]))
        o_r[...] = dinv * _mm(c, w_r[...])

    p1 = _sc_aggregate(g1, src3, dst3, zw[W16], W16)
    (g2,) = _tc(tc_enc, [16], p1, g1, dinv, W2p)
    p2 = _sc_aggregate(g2, src3, dst3, zw[W16], W16)
    (g3,) = _tc(tc_enc, [16], p2, g2, dinv, W3p)
    p3 = _sc_aggregate(g3, src3, dst3, zw[W16], W16)

    def tc_hid(p_r, g_r, dinv_r, h_r, g4_r):
        dinv = dinv_r[...]
        h3 = dinv * (p_r[0] + p_r[1] + g_r[...])
        h_r[...] = h3
        g4_r[...] = dinv * h3
    h3, g4 = _tc(tc_hid, [16, 16], p3, g3, dinv)

    def tc_dec(p_r, g_r, dinv_r, yp_r, lw_r, b_r, tw_r, y_r, gn_r, *, relu):
        dinv = dinv_r[...]
        sh = dinv * (p_r[0] + p_r[1] + g_r[...])
        t = 2.0 * (_mm(yp_r[...], lw_r[...]) + b_r[...]) - _mm(sh, tw_r[...])
        y = jax.nn.relu(t) if relu else t
        y_r[...] = y
        gn_r[...] = dinv * y

    p4 = _sc_aggregate(g4, src3, dst3, zw[W16], W16)
    y1, g5 = _tc(functools.partial(tc_dec, relu=True), [16, 16],
                 p4, g4, dinv, h3, LW1p, b1, TW1p)
    p5 = _sc_aggregate(g5, src3, dst3, zw[W16], W16)
    y2, g6 = _tc(functools.partial(tc_dec, relu=True), [16, 16],
                 p5, g5, dinv, y1, LW2p, b2, TW2p)
    p6 = _sc_aggregate(g6, src3, dst3, zw[W16], W16)

    def tc_out(p_r, g_r, dinv_r, yp_r, lw_r, b_r, tw_r, y_r):
        sh = dinv_r[...] * (p_r[0] + p_r[1] + g_r[...])
        y_r[...] = (2.0 * (_mm(yp_r[...], lw_r[...]) + b_r[...])
                    - _mm(sh, tw_r[...]))
    (y,) = _tc(tc_out, [128], p6, g6, dinv, y2, LW3, b3, TW3)

    return (h3[:NNODE, :4], y[:NNODE])
